# baseline (speedup 1.0000x reference)
import jax
import jax.numpy as jnp
import numpy as np
from jax.experimental import pallas as pl
from jax.experimental.pallas import tpu as pltpu

_BATCH = 64
_N_BINS = 128
_LENGTH = 4096
_MASK_VALUE = -80.0
_MAX_WIDTH = 32


def _rotl(x, r):
    return ((x << np.uint32(r)) | (x >> np.uint32(32 - r))).astype(np.uint32)


def _threefry2x32_pair(k1, k2, c1, c2):
    x = [c1.astype(np.uint32).copy(), c2.astype(np.uint32).copy()]
    rotations = [[13, 15, 26, 6], [17, 29, 16, 24]]
    ks = [np.uint32(k1), np.uint32(k2),
          np.uint32(np.uint32(k1) ^ np.uint32(k2) ^ np.uint32(0x1BD11BDA))]
    x[0] = (x[0] + ks[0]).astype(np.uint32)
    x[1] = (x[1] + ks[1]).astype(np.uint32)
    for i in range(5):
        for r in rotations[i % 2]:
            x[0] = (x[0] + x[1]).astype(np.uint32)
            x[1] = _rotl(x[1], r)
            x[1] = x[1] ^ x[0]
        x[0] = (x[0] + ks[(i + 1) % 3]).astype(np.uint32)
        x[1] = (x[1] + ks[(i + 2) % 3] + np.uint32(i + 1)).astype(np.uint32)
    return x[0], x[1]


def _np_uniform(k1, k2, n, minval, maxval):
    b1, b2 = _threefry2x32_pair(k1, k2, np.zeros(n, np.uint32),
                                np.arange(n, dtype=np.uint32))
    bits = b1 ^ b2
    fb = (bits >> np.uint32(9)) | np.uint32(0x3F800000)
    floats = fb.view(np.float32) - np.float32(1.0)
    r = (floats * np.float32(maxval - minval)
         + np.float32(minval)).astype(np.float32)
    return np.maximum(np.float32(minval), r)


def _mask_bounds() -> tuple[np.ndarray, np.ndarray]:
    b1, b2 = _threefry2x32_pair(np.uint32(0), np.uint32(42),
                                np.zeros(2, np.uint32),
                                np.arange(2, dtype=np.uint32))
    width = _np_uniform(b1[0], b2[0], _BATCH, 0.0, float(_MAX_WIDTH))
    ix = _np_uniform(b1[1], b2[1], _BATCH, 0.0, float(_N_BINS - _MAX_WIDTH))
    start = np.floor(ix).astype(np.int32)
    end = np.floor((ix + width).astype(np.float32)).astype(np.int32)
    return start, end


_START, _END = _mask_bounds()

_BB = 4


def _where_kernel(st_ref, en_ref, x_ref, o_ref):
    b = pl.program_id(0)
    row = jax.lax.broadcasted_iota(jnp.int32, (1, _N_BINS, 1), 1)
    for k in range(_BB):
        msk = (row >= st_ref[_BB * b + k]) & (row < en_ref[_BB * b + k])
        o_ref[k:k + 1] = jnp.where(msk, jnp.float32(_MASK_VALUE),
                                   x_ref[k:k + 1])


@jax.jit
def kernel(x):
    return pl.pallas_call(
        _where_kernel,
        grid_spec=pltpu.PrefetchScalarGridSpec(
            num_scalar_prefetch=2,
            grid=(_BATCH // _BB,),
            in_specs=[pl.BlockSpec((_BB, _N_BINS, _LENGTH),
                                   lambda b, st, en: (b, 0, 0))],
            out_specs=pl.BlockSpec((_BB, _N_BINS, _LENGTH),
                                   lambda b, st, en: (b, 0, 0)),
        ),
        out_shape=jax.ShapeDtypeStruct((_BATCH, _N_BINS, _LENGTH), x.dtype),
    )(jnp.asarray(_START), jnp.asarray(_END), x)
